# baseline (speedup 1.0000x reference)
import functools

import jax
import jax.numpy as jnp
import numpy as np
from jax import lax
from jax.experimental import pallas as pl
from jax.experimental.pallas import tpu as pltpu
from jax.experimental.pallas import tpu_sc as plsc

N_GRAPH, N_NODE, N_FEAT, HIDDEN = 256, 128, 9, 128
N_DEG = 512
N_TOTAL = N_GRAPH * N_NODE
NW = 32
NODES_PER_W = N_TOTAL // NW
CHUNK = 128
N_CHUNKS = NODES_PER_W // CHUNK
WIN = 9


def _sc_gather(x_idx, atom_table, scat, zeros, n_graphs):
    n_chunks = n_graphs // NW
    mesh = plsc.VectorSubcoreMesh(core_axis_name="c", subcore_axis_name="s")

    @functools.partial(
        pl.kernel,
        out_type=jax.ShapeDtypeStruct((n_graphs * N_NODE, HIDDEN), jnp.float32),
        mesh=mesh,
        scratch_types=[
            pltpu.VMEM((WIN * CHUNK,), jnp.int32),
            pltpu.VMEM((WIN * CHUNK,), jnp.int32),
            pltpu.VMEM((2 * CHUNK, HIDDEN), jnp.float32),
            pltpu.VMEM((2 * CHUNK, HIDDEN), jnp.float32),
            pltpu.VMEM_SHARED((16, CHUNK, HIDDEN), jnp.float32),
            pltpu.VMEM((CHUNK, HIDDEN), jnp.float32),
            pltpu.SemaphoreType.DMA,
            pltpu.SemaphoreType.DMA,
            pltpu.SemaphoreType.DMA,
            pltpu.SemaphoreType.DMA,
            pltpu.SemaphoreType.DMA,
            pltpu.SemaphoreType.DMA,
        ],
    )
    def kern(x_hbm, atab_hbm, scat_hbm, zeros_hbm, afeat_hbm,
             idx_v, scat_v, rows0, rows1, accum_sh, zeros_v,
             sem_g0, sem_g1, sem_s0, sem_s1, sem_z, sem_o):
        sid = lax.axis_index("s")
        wid = sid * 2 + lax.axis_index("c")
        accum_v = accum_sh.at[sid]
        rows = (rows0, rows1)
        sem_g = (sem_g0, sem_g1)
        sem_s = (sem_s0, sem_s1)
        wins = [(0, 256), (256, 256), (512, 256), (768, 256), (1024, 128)]
        pltpu.sync_copy(scat_hbm, scat_v)
        pltpu.sync_copy(zeros_hbm, zeros_v)

        @pl.loop(0, n_chunks)
        def _(c):
            gidx = wid * n_chunks + c
            base = gidx * CHUNK
            pltpu.sync_copy(x_hbm.at[gidx], idx_v)
            g = [None] * 2
            s = [None] * 2
            o0, n0 = wins[0]
            g[0] = pltpu.async_copy(
                atab_hbm.at[idx_v.at[pl.ds(o0, n0)]],
                rows[0].at[pl.ds(0, n0)], sem_g[0])

            @pl.when(c > 0)
            def _():
                pltpu.make_async_copy(
                    accum_v, afeat_hbm.at[pl.ds(base, CHUNK)], sem_o).wait()
            z = pltpu.async_copy(zeros_v, accum_v, sem_z)

            for w in range(len(wins)):
                nxt = w + 1
                if nxt < len(wins):
                    b = nxt % 2
                    if s[b] is not None:
                        s[b].wait()
                        s[b] = None
                    o, n = wins[nxt]
                    g[b] = pltpu.async_copy(
                        atab_hbm.at[idx_v.at[pl.ds(o, n)]],
                        rows[b].at[pl.ds(0, n)], sem_g[b])
                g[w % 2].wait()
                if w == 0:
                    z.wait()
                o, n = wins[w]
                s[w % 2] = pltpu.async_copy(
                    rows[w % 2].at[pl.ds(0, n)],
                    accum_v.at[scat_v.at[pl.ds(o, n)]], sem_s[w % 2],
                    add=True)
            for b in range(2):
                if s[b] is not None:
                    s[b].wait()
            pltpu.async_copy(accum_v, afeat_hbm.at[pl.ds(base, CHUNK)], sem_o)

        pltpu.make_async_copy(
            accum_v, afeat_hbm.at[pl.ds(wid * n_chunks * CHUNK, CHUNK)],
            sem_o).wait()

    return kern(x_idx, atom_table, scat, zeros)


def _fold_body(itab, otab, w1b, w1c, pcat):
    pcat[:N_DEG] = jnp.dot(itab[...], w1b[...],
                           preferred_element_type=jnp.float32).astype(jnp.bfloat16)
    pcat[N_DEG:] = jnp.dot(otab[...], w1c[...],
                           preferred_element_type=jnp.float32).astype(jnp.bfloat16)


def _fold_tables(in_tab, out_tab, w1b, w1c):
    p = jax.ShapeDtypeStruct((2 * N_DEG, HIDDEN), jnp.bfloat16)
    return pl.pallas_call(_fold_body, out_shape=p)(in_tab, out_tab, w1b, w1c)


_G_BLK = 16


_DN0 = (((0,), (0,)), ((), ()))


def _mlp_body(a_ref, i_ref, o_ref, w1a, pcat, b1, w2, b2, tok, out_ref):
    rows = _G_BLK * N_NODE
    bf = jnp.bfloat16
    h = jnp.dot(a_ref[...].reshape(rows, HIDDEN).astype(bf),
                w1a[...].astype(bf), preferred_element_type=jnp.float32)
    rowsT = lax.broadcasted_iota(jnp.int32, (N_DEG, N_NODE), 0)
    pin16, pout16 = pcat[:N_DEG], pcat[N_DEG:]
    degs = []
    for k in range(_G_BLK):
        oh_i = (rowsT == i_ref[k][None, :]).astype(bf)
        dk = lax.dot_general(oh_i, pin16, _DN0,
                             preferred_element_type=jnp.float32)
        oh_o = (rowsT == o_ref[k][None, :]).astype(bf)
        dk = dk + lax.dot_general(oh_o, pout16, _DN0,
                                  preferred_element_type=jnp.float32)
        degs.append(dk)
    h = h + jnp.concatenate(degs, axis=0)
    h = jnp.maximum(h + b1[...], 0.0)
    y = jnp.dot(h.astype(bf), w2[...].astype(bf),
                preferred_element_type=jnp.float32) + b2[...]
    y = y.reshape(_G_BLK, N_NODE, HIDDEN)
    out_ref[:, 1:, :] = y
    out_ref[:, 0:1, :] = jnp.broadcast_to(tok[...][None], (_G_BLK, 1, HIDDEN))


def _tc_mlp(afeat, in_deg, out_deg, w1a, pcat, b1, w2, b2, tok,
            n_graphs, off, prev=None):
    full = lambda shape: pl.BlockSpec(shape, lambda g: (0,) * len(shape))
    blk_off = off // _G_BLK
    in_specs = [
        pl.BlockSpec((_G_BLK, N_NODE, HIDDEN), lambda g: (g, 0, 0)),
        pl.BlockSpec((_G_BLK, N_NODE), lambda g: (g, 0)),
        pl.BlockSpec((_G_BLK, N_NODE), lambda g: (g, 0)),
        full((HIDDEN, HIDDEN)),
        full((2 * N_DEG, HIDDEN)),
        full((1, HIDDEN)), full((HIDDEN, HIDDEN)), full((1, HIDDEN)),
        full((1, HIDDEN)),
    ]
    args = [afeat, in_deg, out_deg, w1a, pcat, b1, w2, b2, tok]
    aliases = {}
    body = _mlp_body
    if prev is not None:
        in_specs.append(pl.BlockSpec(memory_space=pl.ANY))
        args.append(prev)
        aliases = {9: 0}
        body = lambda *refs: _mlp_body(*refs[:9], refs[-1])
    return pl.pallas_call(
        body,
        grid=(n_graphs // _G_BLK,),
        in_specs=in_specs,
        out_specs=pl.BlockSpec((_G_BLK, N_NODE + 1, HIDDEN),
                               lambda g: (g + blk_off, 0, 0)),
        out_shape=jax.ShapeDtypeStruct((N_GRAPH, N_NODE + 1, HIDDEN), jnp.float32),
        input_output_aliases=aliases,
    )(*args)


_SCAT = np.arange(WIN * CHUNK, dtype=np.int32) // N_FEAT


def kernel(x, in_degree, out_degree, atom_table, in_deg_table, out_deg_table,
           W1, b1, W2, b2, graph_token):
    x_idx = x.astype(jnp.int32).reshape(N_GRAPH, N_NODE * N_FEAT)
    scat = jnp.asarray(_SCAT)
    zeros = jnp.zeros((CHUNK, HIDDEN), jnp.float32)
    half = N_GRAPH // 2

    afa = _sc_gather(x_idx[:half], atom_table, scat, zeros, half)
    afb = _sc_gather(x_idx[half:], atom_table, scat, zeros, half)

    w1a, w1b, w1c = W1[:HIDDEN], W1[HIDDEN:2 * HIDDEN], W1[2 * HIDDEN:]
    pcat = _fold_tables(in_deg_table, out_deg_table, w1b, w1c)
    ind = in_degree.astype(jnp.int32)
    outd = out_degree.astype(jnp.int32)
    common = (w1a, pcat, b1.reshape(1, HIDDEN), W2, b2.reshape(1, HIDDEN),
              graph_token.reshape(1, HIDDEN))
    out = _tc_mlp(afa.reshape(half, N_NODE, HIDDEN), ind[:half], outd[:half],
                  *common, n_graphs=half, off=0)
    out = _tc_mlp(afb.reshape(half, N_NODE, HIDDEN), ind[half:], outd[half:],
                  *common, n_graphs=half, off=half, prev=out)
    return out


# speedup vs baseline: 1.0558x; 1.0558x over previous
import functools

import jax
import jax.numpy as jnp
import numpy as np
from jax import lax
from jax.experimental import pallas as pl
from jax.experimental.pallas import tpu as pltpu
from jax.experimental.pallas import tpu_sc as plsc

N_GRAPH, N_NODE, N_FEAT, HIDDEN = 256, 128, 9, 128
N_DEG = 512
N_TOTAL = N_GRAPH * N_NODE
NW = 32
NODES_PER_W = N_TOTAL // NW
CHUNK = 128
N_CHUNKS = NODES_PER_W // CHUNK
WIN = 9


def _sc_gather(x_idx, atom_table, scat, zeros, n_graphs):
    n_chunks = n_graphs // NW
    mesh = plsc.VectorSubcoreMesh(core_axis_name="c", subcore_axis_name="s")

    @functools.partial(
        pl.kernel,
        out_type=jax.ShapeDtypeStruct((n_graphs * N_NODE, HIDDEN), jnp.float32),
        mesh=mesh,
        scratch_types=[
            pltpu.VMEM((WIN * CHUNK,), jnp.int32),
            pltpu.VMEM((WIN, CHUNK), jnp.int32),
            pltpu.VMEM((CHUNK, HIDDEN), jnp.float32),
            pltpu.VMEM((CHUNK, HIDDEN), jnp.float32),
            pltpu.VMEM((CHUNK, HIDDEN), jnp.float32),
            pltpu.VMEM((CHUNK, HIDDEN), jnp.float32),
            pltpu.VMEM_SHARED((16, CHUNK, HIDDEN), jnp.float32),
            pltpu.VMEM((CHUNK, HIDDEN), jnp.float32),
            pltpu.SemaphoreType.DMA,
            pltpu.SemaphoreType.DMA,
            pltpu.SemaphoreType.DMA,
            pltpu.SemaphoreType.DMA,
            pltpu.SemaphoreType.DMA,
            pltpu.SemaphoreType.DMA,
            pltpu.SemaphoreType.DMA,
            pltpu.SemaphoreType.DMA,
            pltpu.SemaphoreType.DMA,
            pltpu.SemaphoreType.DMA,
        ],
    )
    def kern(x_hbm, atab_hbm, scat_hbm, zeros_hbm, afeat_hbm,
             idx_v, scat_v, rows0, rows1, rows2, rows3, accum_sh, zeros_v,
             sem_g0, sem_g1, sem_g2, sem_g3,
             sem_s0, sem_s1, sem_s2, sem_s3, sem_z, sem_o):
        sid = lax.axis_index("s")
        wid = sid * 2 + lax.axis_index("c")
        accum_v = accum_sh.at[sid]
        rows = (rows0, rows1, rows2, rows3)
        sem_g = (sem_g0, sem_g1, sem_g2, sem_g3)
        sem_s = (sem_s0, sem_s1, sem_s2, sem_s3)
        pltpu.sync_copy(scat_hbm, scat_v)
        pltpu.sync_copy(zeros_hbm, zeros_v)

        @pl.loop(0, n_chunks)
        def _(c):
            gidx = wid * n_chunks + c
            base = gidx * CHUNK
            pltpu.sync_copy(x_hbm.at[gidx], idx_v)
            g = [None] * 4
            s = [None] * 4
            for w in range(2):
                g[w] = pltpu.async_copy(
                    atab_hbm.at[idx_v.at[pl.ds(w * CHUNK, CHUNK)]],
                    rows[w], sem_g[w])

            @pl.when(c > 0)
            def _():
                pltpu.make_async_copy(
                    accum_v, afeat_hbm.at[pl.ds(base, CHUNK)], sem_o).wait()
            z = pltpu.async_copy(zeros_v, accum_v, sem_z)

            for w in range(WIN):
                nxt = w + 2
                if nxt < WIN:
                    b = nxt % 4
                    if s[b] is not None:
                        s[b].wait()
                        s[b] = None
                    g[b] = pltpu.async_copy(
                        atab_hbm.at[idx_v.at[pl.ds(nxt * CHUNK, CHUNK)]],
                        rows[b], sem_g[b])
                g[w % 4].wait()
                if w == 0:
                    z.wait()
                s[w % 4] = pltpu.async_copy(
                    rows[w % 4], accum_v.at[scat_v.at[w]],
                    sem_s[w % 4], add=True)
            for b in range(4):
                if s[b] is not None:
                    s[b].wait()
            pltpu.async_copy(accum_v, afeat_hbm.at[pl.ds(base, CHUNK)], sem_o)

        pltpu.make_async_copy(
            accum_v, afeat_hbm.at[pl.ds(wid * n_chunks * CHUNK, CHUNK)],
            sem_o).wait()

    return kern(x_idx, atom_table, scat, zeros)


def _fold_body(itab, otab, w1b, w1c, pcat):
    pcat[:N_DEG] = jnp.dot(itab[...], w1b[...],
                           preferred_element_type=jnp.float32).astype(jnp.bfloat16)
    pcat[N_DEG:] = jnp.dot(otab[...], w1c[...],
                           preferred_element_type=jnp.float32).astype(jnp.bfloat16)


def _fold_tables(in_tab, out_tab, w1b, w1c):
    p = jax.ShapeDtypeStruct((2 * N_DEG, HIDDEN), jnp.bfloat16)
    return pl.pallas_call(_fold_body, out_shape=p)(in_tab, out_tab, w1b, w1c)


_G_BLK = 16


_DN0 = (((0,), (0,)), ((), ()))


def _mlp_body(a_ref, i_ref, o_ref, w1a, pcat, b1, w2, b2, tok, out_ref):
    rows = _G_BLK * N_NODE
    bf = jnp.bfloat16
    h = jnp.dot(a_ref[...].reshape(rows, HIDDEN).astype(bf),
                w1a[...].astype(bf), preferred_element_type=jnp.float32)
    rowsT = lax.broadcasted_iota(jnp.int32, (N_DEG, N_NODE), 0)
    pin16, pout16 = pcat[:N_DEG], pcat[N_DEG:]
    degs = []
    for k in range(_G_BLK):
        oh_i = (rowsT == i_ref[k][None, :]).astype(bf)
        dk = lax.dot_general(oh_i, pin16, _DN0,
                             preferred_element_type=jnp.float32)
        oh_o = (rowsT == o_ref[k][None, :]).astype(bf)
        dk = dk + lax.dot_general(oh_o, pout16, _DN0,
                                  preferred_element_type=jnp.float32)
        degs.append(dk)
    h = h + jnp.concatenate(degs, axis=0)
    h = jnp.maximum(h + b1[...], 0.0)
    y = jnp.dot(h.astype(bf), w2[...].astype(bf),
                preferred_element_type=jnp.float32) + b2[...]
    y = y.reshape(_G_BLK, N_NODE, HIDDEN)
    out_ref[:, 1:, :] = y
    out_ref[:, 0:1, :] = jnp.broadcast_to(tok[...][None], (_G_BLK, 1, HIDDEN))


def _tc_mlp(afeat, in_deg, out_deg, w1a, pcat, b1, w2, b2, tok,
            n_graphs, off, prev=None):
    full = lambda shape: pl.BlockSpec(shape, lambda g: (0,) * len(shape))
    blk_off = off // _G_BLK
    in_specs = [
        pl.BlockSpec((_G_BLK, N_NODE, HIDDEN), lambda g: (g, 0, 0)),
        pl.BlockSpec((_G_BLK, N_NODE), lambda g: (g, 0)),
        pl.BlockSpec((_G_BLK, N_NODE), lambda g: (g, 0)),
        full((HIDDEN, HIDDEN)),
        full((2 * N_DEG, HIDDEN)),
        full((1, HIDDEN)), full((HIDDEN, HIDDEN)), full((1, HIDDEN)),
        full((1, HIDDEN)),
    ]
    args = [afeat, in_deg, out_deg, w1a, pcat, b1, w2, b2, tok]
    aliases = {}
    body = _mlp_body
    if prev is not None:
        in_specs.append(pl.BlockSpec(memory_space=pl.ANY))
        args.append(prev)
        aliases = {9: 0}
        body = lambda *refs: _mlp_body(*refs[:9], refs[-1])
    return pl.pallas_call(
        body,
        grid=(n_graphs // _G_BLK,),
        in_specs=in_specs,
        out_specs=pl.BlockSpec((_G_BLK, N_NODE + 1, HIDDEN),
                               lambda g: (g + blk_off, 0, 0)),
        out_shape=jax.ShapeDtypeStruct((N_GRAPH, N_NODE + 1, HIDDEN), jnp.float32),
        input_output_aliases=aliases,
    )(*args)


_SCAT = np.arange(WIN * CHUNK, dtype=np.int32).reshape(WIN, CHUNK) // N_FEAT


def kernel(x, in_degree, out_degree, atom_table, in_deg_table, out_deg_table,
           W1, b1, W2, b2, graph_token):
    x_idx = x.astype(jnp.int32).reshape(N_GRAPH, N_NODE * N_FEAT)
    scat = jnp.asarray(_SCAT)
    zeros = jnp.zeros((CHUNK, HIDDEN), jnp.float32)
    half = N_GRAPH // 2

    afa = _sc_gather(x_idx[:half], atom_table, scat, zeros, half)
    afb = _sc_gather(x_idx[half:], atom_table, scat, zeros, half)

    w1a, w1b, w1c = W1[:HIDDEN], W1[HIDDEN:2 * HIDDEN], W1[2 * HIDDEN:]
    pcat = _fold_tables(in_deg_table, out_deg_table, w1b, w1c)
    ind = in_degree.astype(jnp.int32)
    outd = out_degree.astype(jnp.int32)
    common = (w1a, pcat, b1.reshape(1, HIDDEN), W2, b2.reshape(1, HIDDEN),
              graph_token.reshape(1, HIDDEN))
    out = _tc_mlp(afa.reshape(half, N_NODE, HIDDEN), ind[:half], outd[:half],
                  *common, n_graphs=half, off=0)
    out = _tc_mlp(afb.reshape(half, N_NODE, HIDDEN), ind[half:], outd[half:],
                  *common, n_graphs=half, off=half, prev=out)
    return out
